# baseline (speedup 1.0000x reference)
import functools

import jax
import jax.numpy as jnp
from jax import lax
from jax.experimental import pallas as pl
from jax.experimental.pallas import tpu as pltpu
from jax.experimental.pallas import tpu_sc as plsc

BATCH = 4096
SEQ = 200
DIM = 64

_info = plsc.get_sparse_core_info()
NC, NS, NL = _info.num_cores, _info.num_subcores, _info.num_lanes
NW = NC * NS
ROWS_PER_W = BATCH // NW
CH = 2
CHUNK = CH * SEQ
NBUF = 4
G = ROWS_PER_W // CH

BB = 1024
SP = SEQ // 2


def _sc_gather(idx_hbm, tok_hbm, out_hbm, idx_bufs, row_bufs, in_sems,
               out_sems):
    wid = lax.axis_index("s") * NC + lax.axis_index("c")
    wbase = wid * (ROWS_PER_W * SEQ)

    def start_gather(g, b):
        base = wbase + g * CHUNK
        pltpu.sync_copy(idx_hbm.at[pl.ds(base, CHUNK)], idx_bufs[b])
        pltpu.async_copy(tok_hbm.at[idx_bufs[b]], row_bufs[b], in_sems[b])

    for b in range(NBUF - 1):
        start_gather(b, b)

    def outer(k, _):
        for b in range(NBUF):
            g = k * NBUF + b
            pltpu.make_async_copy(tok_hbm.at[idx_bufs[b]], row_bufs[b],
                                  in_sems[b]).wait()

            bp = (b + NBUF - 1) % NBUF

            @pl.when(jnp.logical_and(g >= 1, g + NBUF - 1 < G))
            def _():
                pltpu.make_async_copy(row_bufs[bp],
                                      out_hbm.at[pl.ds(0, CHUNK)],
                                      out_sems[bp]).wait()

            @pl.when(g + NBUF - 1 < G)
            def _():
                start_gather(g + NBUF - 1, bp)

            pltpu.async_copy(row_bufs[b],
                             out_hbm.at[pl.ds(wbase + g * CHUNK, CHUNK)],
                             out_sems[b])
        return 0

    lax.fori_loop(0, G // NBUF, outer, 0)

    for b in range(NBUF):
        pltpu.make_async_copy(row_bufs[b], out_hbm.at[pl.ds(0, CHUNK)],
                              out_sems[b]).wait()


def _tc_relayout(rows_ref, pos_ref, out_ref):
    x = rows_ref[...]
    p = pos_ref[0]
    for h in range(2):
        xt = jnp.swapaxes(x[:, h * DIM:(h + 1) * DIM], 0, 1)
        out_ref[h] = xt + p[h][:, None]


@jax.jit
def kernel(inputs, token_table, position_table):
    idx_flat = inputs.reshape(-1).astype(jnp.int32)
    mesh = plsc.VectorSubcoreMesh(core_axis_name="c", subcore_axis_name="s")
    gathered = pl.kernel(
        _sc_gather,
        mesh=mesh,
        out_type=jax.ShapeDtypeStruct((BATCH * SEQ, DIM), jnp.float32),
        scratch_types=[
            [pltpu.VMEM((CHUNK,), jnp.int32) for _ in range(NBUF)],
            [pltpu.VMEM((CHUNK, DIM), jnp.float32) for _ in range(NBUF)],
            [pltpu.SemaphoreType.DMA for _ in range(NBUF)],
            [pltpu.SemaphoreType.DMA for _ in range(NBUF)],
        ],
        compiler_params=pltpu.CompilerParams(use_tc_tiling_on_sc=False),
    )(idx_flat, token_table)

    rows = gathered.reshape(BATCH, SP * 128)
    pos = position_table.reshape(SP, 2, DIM)
    out_phys = pl.pallas_call(
        _tc_relayout,
        grid=(SP, BATCH // BB),
        in_specs=[
            pl.BlockSpec((BB, 128), lambda sp, bb: (bb, sp)),
            pl.BlockSpec((1, 2, DIM), lambda sp, bb: (sp, 0, 0)),
        ],
        out_specs=pl.BlockSpec((2, DIM, BB), lambda sp, bb: (sp, 0, bb)),
        out_shape=jax.ShapeDtypeStruct((SEQ, DIM, BATCH), jnp.float32),
    )(rows, pos)
    return out_phys.transpose(2, 0, 1)


# speedup vs baseline: 1.4055x; 1.4055x over previous
import functools

import jax
import jax.numpy as jnp
from jax import lax
from jax.experimental import pallas as pl
from jax.experimental.pallas import tpu as pltpu
from jax.experimental.pallas import tpu_sc as plsc

BATCH = 4096
SEQ = 200
DIM = 64

_info = plsc.get_sparse_core_info()
NC, NS, NL = _info.num_cores, _info.num_subcores, _info.num_lanes
NW = NC * NS
GROUPS_PER_W = BATCH // (8 * NW)
HALF = SEQ // 4
CHUNK = HALF * 16

BB = 1024
SP = SEQ // 2


def _sc_gather(idx_hbm, tok_hbm, out_hbm, raw_bufs, idxp_bufs, row_bufs,
               raw_sems, in_sems, out_sems):
    wid = lax.axis_index("s") * NC + lax.axis_index("c")

    lanes = lax.iota(jnp.int32, NL)
    offs_base = (lanes // 2) * SEQ + (lanes % 2)

    def raw_base(gi):
        return (wid * GROUPS_PER_W + gi) * (8 * SEQ)

    pltpu.async_copy(idx_hbm.at[pl.ds(raw_base(0), 8 * SEQ)], raw_bufs[0],
                     raw_sems[0])

    def pair(k, _):
        for par in range(2):
            gi = k * 2 + par
            gb = par
            pltpu.make_async_copy(idx_hbm.at[pl.ds(0, 8 * SEQ)], raw_bufs[gb],
                                  raw_sems[gb]).wait()

            @pl.when(gi + 1 < GROUPS_PER_W)
            def _():
                pltpu.async_copy(idx_hbm.at[pl.ds(raw_base(gi + 1), 8 * SEQ)],
                                 raw_bufs[1 - gb], raw_sems[1 - gb])

            for hh in range(2):
                raw = raw_bufs[gb]
                idxp = idxp_bufs[hh]

                @plsc.parallel_loop(0, HALF, unroll=4)
                def _(i):
                    offs = offs_base + (2 * (hh * HALF) + 2 * i)
                    idxp[pl.ds(i * NL, NL)] = plsc.load_gather(raw, [offs])

            for hh in range(2):
                @pl.when(gi >= 1)
                def _():
                    pltpu.make_async_copy(row_bufs[hh],
                                          out_hbm.at[pl.ds(0, CHUNK)],
                                          out_sems[hh]).wait()

                pltpu.async_copy(tok_hbm.at[idxp_bufs[hh]], row_bufs[hh],
                                 in_sems[hh])

            for hh in range(2):
                c = (wid * GROUPS_PER_W + gi) * 2 + hh
                pltpu.make_async_copy(tok_hbm.at[idxp_bufs[hh]], row_bufs[hh],
                                      in_sems[hh]).wait()
                pltpu.async_copy(row_bufs[hh],
                                 out_hbm.at[pl.ds(c * CHUNK, CHUNK)],
                                 out_sems[hh])
        return 0

    lax.fori_loop(0, GROUPS_PER_W // 2, pair, 0)

    for hh in range(2):
        pltpu.make_async_copy(row_bufs[hh], out_hbm.at[pl.ds(0, CHUNK)],
                              out_sems[hh]).wait()


def _tc_relayout(rows_ref, pos_ref, out_ref):
    x = rows_ref[...].reshape(BB, 128)
    p = pos_ref[0]
    eye = (lax.broadcasted_iota(jnp.int32, (DIM, DIM), 0) ==
           lax.broadcasted_iota(jnp.int32, (DIM, DIM), 1)).astype(jnp.float32)
    for h in range(2):
        xh = x[:, h * DIM:(h + 1) * DIM]
        xt = lax.dot_general(eye, xh, (((1,), (1,)), ((), ())),
                             preferred_element_type=jnp.float32)
        out_ref[h] = xt + p[h][:, None]


@jax.jit
def kernel(inputs, token_table, position_table):
    idx_flat = inputs.reshape(-1).astype(jnp.int32)
    mesh = plsc.VectorSubcoreMesh(core_axis_name="c", subcore_axis_name="s")
    gathered = pl.kernel(
        _sc_gather,
        mesh=mesh,
        out_type=jax.ShapeDtypeStruct((BATCH * SEQ, DIM), jnp.float32),
        scratch_types=[
            [pltpu.VMEM((8 * SEQ,), jnp.int32) for _ in range(2)],
            [pltpu.VMEM((CHUNK,), jnp.int32) for _ in range(2)],
            [pltpu.VMEM((CHUNK, DIM), jnp.float32) for _ in range(2)],
            [pltpu.SemaphoreType.DMA for _ in range(2)],
            [pltpu.SemaphoreType.DMA for _ in range(2)],
            [pltpu.SemaphoreType.DMA for _ in range(2)],
        ],
        compiler_params=pltpu.CompilerParams(use_tc_tiling_on_sc=False, needs_layout_passes=False),
    )(idx_flat, token_table)

    rows = gathered.reshape(BATCH // 8, SP, 8, 128)
    pos = position_table.reshape(SP, 2, DIM)
    out_phys = pl.pallas_call(
        _tc_relayout,
        grid=(SP, BATCH // BB),
        in_specs=[
            pl.BlockSpec((BB // 8, 1, 8, 128), lambda sp, bb: (bb, sp, 0, 0)),
            pl.BlockSpec((1, 2, DIM), lambda sp, bb: (sp, 0, 0)),
        ],
        out_specs=pl.BlockSpec((2, DIM, BB), lambda sp, bb: (sp, 0, bb)),
        out_shape=jax.ShapeDtypeStruct((SEQ, DIM, BATCH), jnp.float32),
    )(rows, pos)
    return out_phys.transpose(2, 0, 1)


# speedup vs baseline: 2.1383x; 1.5214x over previous
import functools

import jax
import jax.numpy as jnp
from jax import lax
from jax.experimental import pallas as pl
from jax.experimental.pallas import tpu as pltpu
from jax.experimental.pallas import tpu_sc as plsc

BATCH = 4096
SEQ = 200
DIM = 64

_info = plsc.get_sparse_core_info()
NC, NS, NL = _info.num_cores, _info.num_subcores, _info.num_lanes
NW = NC * NS
GROUPS_PER_W = BATCH // (8 * NW)
HALF = SEQ // 4
CHUNK = HALF * 16

BB = 4096
KSP = 2
SP = SEQ // 2


def _sc_gather(idx_hbm, tok_hbm, out_hbm, raw_bufs, idxp_bufs, row_bufs,
               raw_sems, in_sems, out_sems):
    wid = lax.axis_index("s") * NC + lax.axis_index("c")

    lanes = lax.iota(jnp.int32, NL)
    offs_base = (lanes // 2) * SEQ + (lanes % 2)

    def raw_base(gi):
        return (wid * GROUPS_PER_W + gi) * (8 * SEQ)

    pltpu.async_copy(idx_hbm.at[pl.ds(raw_base(0), 8 * SEQ)], raw_bufs[0],
                     raw_sems[0])

    def pair(k, _):
        for par in range(2):
            gi = k * 2 + par
            gb = par
            pltpu.make_async_copy(idx_hbm.at[pl.ds(0, 8 * SEQ)], raw_bufs[gb],
                                  raw_sems[gb]).wait()

            @pl.when(gi + 1 < GROUPS_PER_W)
            def _():
                pltpu.async_copy(idx_hbm.at[pl.ds(raw_base(gi + 1), 8 * SEQ)],
                                 raw_bufs[1 - gb], raw_sems[1 - gb])

            for hh in range(2):
                raw = raw_bufs[gb]
                idxp = idxp_bufs[hh]

                @plsc.parallel_loop(0, HALF, unroll=4)
                def _(i):
                    offs = offs_base + (2 * (hh * HALF) + 2 * i)
                    idxp[pl.ds(i * NL, NL)] = plsc.load_gather(raw, [offs])

            for hh in range(2):
                @pl.when(gi >= 1)
                def _():
                    pltpu.make_async_copy(row_bufs[hh],
                                          out_hbm.at[pl.ds(0, CHUNK)],
                                          out_sems[hh]).wait()

                pltpu.async_copy(tok_hbm.at[idxp_bufs[hh]], row_bufs[hh],
                                 in_sems[hh])

            for hh in range(2):
                c = (wid * GROUPS_PER_W + gi) * 2 + hh
                pltpu.make_async_copy(tok_hbm.at[idxp_bufs[hh]], row_bufs[hh],
                                      in_sems[hh]).wait()
                pltpu.async_copy(row_bufs[hh],
                                 out_hbm.at[pl.ds(c * CHUNK, CHUNK)],
                                 out_sems[hh])
        return 0

    lax.fori_loop(0, GROUPS_PER_W // 2, pair, 0)

    for hh in range(2):
        pltpu.make_async_copy(row_bufs[hh], out_hbm.at[pl.ds(0, CHUNK)],
                              out_sems[hh]).wait()


def _tc_relayout(rows_ref, pos_ref, out_ref):
    x4 = rows_ref[...]
    p = pos_ref[...]
    eye = (lax.broadcasted_iota(jnp.int32, (DIM, DIM), 0) ==
           lax.broadcasted_iota(jnp.int32, (DIM, DIM), 1)).astype(jnp.float32)
    for j in range(KSP):
        x = x4[:, j].reshape(BB, 128)
        for h in range(2):
            xh = x[:, h * DIM:(h + 1) * DIM]
            xt = lax.dot_general(eye, xh, (((1,), (1,)), ((), ())),
                                 preferred_element_type=jnp.float32)
            out_ref[2 * j + h] = xt + p[j, h][:, None]


@jax.jit
def kernel(inputs, token_table, position_table):
    idx_flat = inputs.reshape(-1).astype(jnp.int32)
    mesh = plsc.VectorSubcoreMesh(core_axis_name="c", subcore_axis_name="s")
    gathered = pl.kernel(
        _sc_gather,
        mesh=mesh,
        out_type=jax.ShapeDtypeStruct((BATCH * SEQ, DIM), jnp.float32),
        scratch_types=[
            [pltpu.VMEM((8 * SEQ,), jnp.int32) for _ in range(2)],
            [pltpu.VMEM((CHUNK,), jnp.int32) for _ in range(2)],
            [pltpu.VMEM((CHUNK, DIM), jnp.float32) for _ in range(2)],
            [pltpu.SemaphoreType.DMA for _ in range(2)],
            [pltpu.SemaphoreType.DMA for _ in range(2)],
            [pltpu.SemaphoreType.DMA for _ in range(2)],
        ],
        compiler_params=pltpu.CompilerParams(use_tc_tiling_on_sc=False, needs_layout_passes=False),
    )(idx_flat, token_table)

    rows = gathered.reshape(BATCH // 8, SP, 8, 128)
    pos = position_table.reshape(SP, 2, DIM)
    out_phys = pl.pallas_call(
        _tc_relayout,
        grid=(SP // KSP,),
        in_specs=[
            pl.BlockSpec((BB // 8, KSP, 8, 128), lambda i: (0, i, 0, 0)),
            pl.BlockSpec((KSP, 2, DIM), lambda i: (i, 0, 0)),
        ],
        out_specs=pl.BlockSpec((2 * KSP, DIM, BB), lambda i: (i, 0, 0)),
        out_shape=jax.ShapeDtypeStruct((SEQ, DIM, BATCH), jnp.float32),
    )(rows, pos)
    return out_phys.transpose(2, 0, 1)


# speedup vs baseline: 2.1845x; 1.0216x over previous
import functools

import jax
import jax.numpy as jnp
from jax import lax
from jax.experimental import pallas as pl
from jax.experimental.pallas import tpu as pltpu
from jax.experimental.pallas import tpu_sc as plsc

BATCH = 4096
SEQ = 200
DIM = 64

_info = plsc.get_sparse_core_info()
NC, NS, NL = _info.num_cores, _info.num_subcores, _info.num_lanes
NW = NC * NS
GROUPS_PER_W = BATCH // (8 * NW)
NQ = 4
QSP = SEQ // (2 * NQ)
QCHUNK = QSP * 16

BB = 4096
KSP = 5
SP = SEQ // 2


def _sc_gather(idx_hbm, tok_hbm, out_hbm, raw_bufs, idxp_bufs, row_bufs,
               raw_sems, in_sems, out_sems):
    wid = lax.axis_index("s") * NC + lax.axis_index("c")

    lanes = lax.iota(jnp.int32, NL)
    offs_base = (lanes // 2) * SEQ + (lanes % 2)

    def raw_base(gi):
        return (wid * GROUPS_PER_W + gi) * (8 * SEQ)

    pltpu.async_copy(idx_hbm.at[pl.ds(raw_base(0), 8 * SEQ)], raw_bufs[0],
                     raw_sems[0])

    def pair(k, _):
        for par in range(2):
            gi = k * 2 + par
            gb = par
            pltpu.make_async_copy(idx_hbm.at[pl.ds(0, 8 * SEQ)], raw_bufs[gb],
                                  raw_sems[gb]).wait()

            @pl.when(gi + 1 < GROUPS_PER_W)
            def _():
                pltpu.async_copy(idx_hbm.at[pl.ds(raw_base(gi + 1), 8 * SEQ)],
                                 raw_bufs[1 - gb], raw_sems[1 - gb])

            for q in range(NQ):
                raw = raw_bufs[gb]
                idxp = idxp_bufs[q]

                @plsc.parallel_loop(0, QSP, unroll=4)
                def _(i):
                    offs = offs_base + (2 * (q * QSP) + 2 * i)
                    idxp[pl.ds(i * NL, NL)] = plsc.load_gather(raw, [offs])

            for q in range(NQ):
                @pl.when(gi >= 1)
                def _():
                    pltpu.make_async_copy(row_bufs[q],
                                          out_hbm.at[pl.ds(0, QCHUNK)],
                                          out_sems[q]).wait()

                pltpu.async_copy(tok_hbm.at[idxp_bufs[q]], row_bufs[q],
                                 in_sems[q])

            for q in range(NQ):
                c = (wid * GROUPS_PER_W + gi) * NQ + q
                pltpu.make_async_copy(tok_hbm.at[idxp_bufs[q]], row_bufs[q],
                                      in_sems[q]).wait()
                pltpu.async_copy(row_bufs[q],
                                 out_hbm.at[pl.ds(c * QCHUNK, QCHUNK)],
                                 out_sems[q])
        return 0

    lax.fori_loop(0, GROUPS_PER_W // 2, pair, 0)

    for q in range(NQ):
        pltpu.make_async_copy(row_bufs[q], out_hbm.at[pl.ds(0, QCHUNK)],
                              out_sems[q]).wait()


def _tc_relayout(rows_ref, pos_ref, out_ref):
    x4 = rows_ref[...]
    p = pos_ref[...]
    eye = (lax.broadcasted_iota(jnp.int32, (DIM, DIM), 0) ==
           lax.broadcasted_iota(jnp.int32, (DIM, DIM), 1)).astype(jnp.float32)
    for j in range(KSP):
        x = x4[:, j].reshape(BB, 128)
        for h in range(2):
            xh = x[:, h * DIM:(h + 1) * DIM]
            xt = lax.dot_general(eye, xh, (((1,), (1,)), ((), ())),
                                 preferred_element_type=jnp.float32)
            out_ref[2 * j + h] = xt + p[j, h][:, None]


@jax.jit
def kernel(inputs, token_table, position_table):
    idx_flat = inputs.reshape(-1).astype(jnp.int32)
    mesh = plsc.VectorSubcoreMesh(core_axis_name="c", subcore_axis_name="s")
    gathered = pl.kernel(
        _sc_gather,
        mesh=mesh,
        out_type=jax.ShapeDtypeStruct((BATCH * SEQ, DIM), jnp.float32),
        scratch_types=[
            [pltpu.VMEM((8 * SEQ,), jnp.int32) for _ in range(2)],
            [pltpu.VMEM((QCHUNK,), jnp.int32) for _ in range(NQ)],
            [pltpu.VMEM((QCHUNK, DIM), jnp.float32) for _ in range(NQ)],
            [pltpu.SemaphoreType.DMA for _ in range(2)],
            [pltpu.SemaphoreType.DMA for _ in range(NQ)],
            [pltpu.SemaphoreType.DMA for _ in range(NQ)],
        ],
        compiler_params=pltpu.CompilerParams(use_tc_tiling_on_sc=False,
                                             needs_layout_passes=False),
    )(idx_flat, token_table)

    rows = gathered.reshape(BATCH // 8, SP, 8, 128)
    pos = position_table.reshape(SP, 2, DIM)
    out_phys = pl.pallas_call(
        _tc_relayout,
        grid=(SP // KSP,),
        in_specs=[
            pl.BlockSpec((BB // 8, KSP, 8, 128), lambda i: (0, i, 0, 0)),
            pl.BlockSpec((KSP, 2, DIM), lambda i: (i, 0, 0)),
        ],
        out_specs=pl.BlockSpec((2 * KSP, DIM, BB), lambda i: (i, 0, 0)),
        out_shape=jax.ShapeDtypeStruct((SEQ, DIM, BATCH), jnp.float32),
    )(rows, pos)
    return out_phys.transpose(2, 0, 1)
